# baseline (speedup 1.0000x reference)
import functools

import jax
import jax.numpy as jnp
from jax import lax
from jax.experimental import pallas as pl
from jax.experimental.pallas import tpu as pltpu
from jax.experimental.pallas import tpu_sc as plsc

B = 4096
F = 26
V = 100000
D = 16
NUM = 13

NC = 2
NS = 16
NW = NC * NS
BF = B * F
CPW = BF // NW
TPC = 128
CH = CPW // TPC

@functools.lru_cache(maxsize=1)
def _make_sc_gather():
    mesh = plsc.VectorSubcoreMesh(core_axis_name="c", subcore_axis_name="s")

    @functools.partial(
        pl.kernel,
        out_type=[
            jax.ShapeDtypeStruct((NW, CH, TPC), jnp.float32),
            jax.ShapeDtypeStruct((BF, D), jnp.float32),
            jax.ShapeDtypeStruct((BF, D), jnp.float32),
        ],
        mesh=mesh,
        scratch_types=[
            pltpu.VMEM((CH, TPC), jnp.int32),
            pltpu.VMEM((CH, TPC), jnp.float32),
            pltpu.VMEM((CPW, D), jnp.float32),
            pltpu.VMEM((CPW, D), jnp.float32),
            pltpu.SemaphoreType.DMA,
        ],
        compiler_params=pltpu.CompilerParams(use_tc_tiling_on_sc=False),
    )
    def _sc_gather(idx_hbm, fm1_hbm, fm2_hbm, dnn_hbm,
                   fm1_out, fm2_out, dnn_out,
                   idx_v, fm1_v, fm2_v, dnn_v, sem):
        wid = lax.axis_index("s") * NC + lax.axis_index("c")
        base = wid * CPW
        pltpu.sync_copy(idx_hbm.at[wid], idx_v)

        def issue(j, carry):
            pltpu.async_copy(fm2_hbm.at[idx_v.at[j]], fm2_v.at[pl.ds(j * TPC, TPC)], sem)
            pltpu.async_copy(dnn_hbm.at[idx_v.at[j]], dnn_v.at[pl.ds(j * TPC, TPC)], sem)
            pltpu.async_copy(fm1_hbm.at[idx_v.at[j]], fm1_v.at[j], sem)
            return carry

        lax.fori_loop(0, CH, issue, 0)

        def drain(j, carry):
            pltpu.make_async_copy(fm2_hbm.at[idx_v.at[j]], fm2_v.at[pl.ds(j * TPC, TPC)], sem).wait()
            pltpu.make_async_copy(dnn_hbm.at[idx_v.at[j]], dnn_v.at[pl.ds(j * TPC, TPC)], sem).wait()
            pltpu.make_async_copy(fm1_hbm.at[idx_v.at[j]], fm1_v.at[j], sem).wait()
            return carry

        lax.fori_loop(0, CH, drain, 0)

        pltpu.sync_copy(fm1_v, fm1_out.at[wid])
        pltpu.sync_copy(fm2_v, fm2_out.at[pl.ds(base, CPW)])
        pltpu.sync_copy(dnn_v, dnn_out.at[pl.ds(base, CPW)])

    return _sc_gather


BM = 512


def _tc_body(dnn_ref, fm2_ref, fm1_ref, num_ref,
             w0_ref, b0_ref, w1_ref, b1_ref, w2_ref, b2_ref,
             nw_ref, nb_ref, nvec_ref, woh_ref, par_ref, out_ref):
    fm1s = jnp.sum(fm1_ref[...], axis=1)
    fm2 = fm2_ref[...]
    s = jnp.sum(fm2, axis=1)
    ss = jnp.sum(fm2 * fm2, axis=1)
    fm2nd = 0.5 * (s * s - ss)
    x = num_ref[...]
    ne = jnp.dot(x, nw_ref[...], preferred_element_type=jnp.float32) + nb_ref[...]
    h = jnp.concatenate([dnn_ref[...], ne], axis=1)
    h = jnp.maximum(jnp.dot(h, w0_ref[...], preferred_element_type=jnp.float32) + b0_ref[...], 0.0)
    h = jnp.maximum(jnp.dot(h, w1_ref[...], preferred_element_type=jnp.float32) + b1_ref[...], 0.0)
    h = jnp.maximum(jnp.dot(h, w2_ref[...], preferred_element_type=jnp.float32) + b2_ref[...], 0.0)
    lin = (jnp.dot(h, woh_ref[...], preferred_element_type=jnp.float32)[:, 0]
           + jnp.dot(x, nvec_ref[...], preferred_element_type=jnp.float32)[:, 0])
    logit = par_ref[0, 0] * fm1s + par_ref[0, 1] * fm2nd + lin + par_ref[0, 2]
    out_ref[...] = 1.0 / (1.0 + jnp.exp(-logit))


def _full(shape):
    return pl.BlockSpec(shape, lambda i: tuple(0 for _ in shape))


def kernel(numeric, categorical, fm1_cat, fm1_num_w, fm1_num_b, fm2_tables,
           dnn_tables, num_w, num_b,
           W0, b0, g0, be0, rm0, rv0,
           W1, b1, g1, be1, rm1, rv1,
           W2, b2, g2, be2, rm2, rv2,
           Wo, bo):
    offsets = (jnp.arange(F, dtype=jnp.int32) * V)[None, :]
    flat_idx = (categorical + offsets).reshape(NW, CH, TPC)
    fm1_flat = fm1_cat.reshape(F * V)
    fm2_flat = fm2_tables.reshape(F * V, D)
    dnn_flat = dnn_tables.reshape(F * V, D)

    fm1_vals, fm2_rows, dnn_rows = _make_sc_gather()(flat_idx, fm1_flat, fm2_flat, dnn_flat)
    fm1_vals = fm1_vals.reshape(B, F)
    fm2_e = fm2_rows.reshape(B, F * D)
    dnn_e = dnn_rows.reshape(B, F * D)

    eps = 1e-5

    def fold(W, b, g, be, rm, rv):
        sc = g / jnp.sqrt(rv + eps)
        return W.T * sc[None, :], ((b - rm) * sc + be)[None, :]

    W0T, b0f = fold(W0, b0, g0, be0, rm0, rv0)
    W1T, b1f = fold(W1, b1, g1, be1, rm1, rv1)
    W2T, b2f = fold(W2, b2, g2, be2, rm2, rv2)
    nwT = num_w.T
    nbf = num_b[None, :]
    wo0 = Wo[0, 0]
    nvec = fm1_num_w.T * wo0
    woh = Wo[0, 2:][:, None]
    params = jnp.stack([wo0, Wo[0, 1], bo[0] + wo0 * fm1_num_b[0]])[None, :]

    grid = (B // BM,)
    out = pl.pallas_call(
        _tc_body,
        grid=grid,
        in_specs=[
            pl.BlockSpec((BM, F * D), lambda i: (i, 0)),
            pl.BlockSpec((BM, F * D), lambda i: (i, 0)),
            pl.BlockSpec((BM, F), lambda i: (i, 0)),
            pl.BlockSpec((BM, NUM), lambda i: (i, 0)),
            _full(W0T.shape), _full(b0f.shape),
            _full(W1T.shape), _full(b1f.shape),
            _full(W2T.shape), _full(b2f.shape),
            _full(nwT.shape), _full(nbf.shape),
            _full(nvec.shape), _full(woh.shape), _full(params.shape),
        ],
        out_specs=pl.BlockSpec((BM,), lambda i: (i,)),
        out_shape=jax.ShapeDtypeStruct((B,), jnp.float32),
    )(dnn_e, fm2_e, fm1_vals, numeric,
      W0T, b0f, W1T, b1f, W2T, b2f, nwT, nbf, nvec, woh, params)
    return out


# speedup vs baseline: 2.9048x; 2.9048x over previous
import functools

import jax
import jax.numpy as jnp
from jax import lax
from jax.experimental import pallas as pl
from jax.experimental.pallas import tpu as pltpu
from jax.experimental.pallas import tpu_sc as plsc

B = 4096
F = 26
V = 100000
D = 16
NUM = 13

NC = 2
NS = 16
NW = NC * NS
BPW = B // NW


@functools.lru_cache(maxsize=1)
def _make_sc_gather():
    mesh = plsc.VectorSubcoreMesh(core_axis_name="c", subcore_axis_name="s")

    @functools.partial(
        pl.kernel,
        out_type=[
            jax.ShapeDtypeStruct((F, NW, BPW), jnp.float32),
            jax.ShapeDtypeStruct((F, D, NW, BPW), jnp.float32),
            jax.ShapeDtypeStruct((F, D, NW, BPW), jnp.float32),
        ],
        mesh=mesh,
        scratch_types=[
            pltpu.VMEM((F, BPW), jnp.int32),
            pltpu.VMEM((F, BPW), jnp.float32),
            pltpu.VMEM((F, D, BPW), jnp.float32),
            pltpu.VMEM((F, D, BPW), jnp.float32),
            pltpu.SemaphoreType.DMA,
        ],
        compiler_params=pltpu.CompilerParams(use_tc_tiling_on_sc=False),
    )
    def _sc_gather(cat_hbm, fm1_hbm, fm2_hbm, dnn_hbm,
                   fm1_out, fm2_out, dnn_out,
                   idx_v, fm1_v, fm2_v, dnn_v, sem):
        wid = lax.axis_index("s") * NC + lax.axis_index("c")
        pltpu.sync_copy(cat_hbm.at[:, pl.ds(wid * BPW, BPW)], idx_v)

        def issue(f):
            for d in range(D):
                pltpu.async_copy(fm2_hbm.at[f * D + d].at[idx_v.at[f]],
                                 fm2_v.at[f].at[d], sem)
                pltpu.async_copy(dnn_hbm.at[f * D + d].at[idx_v.at[f]],
                                 dnn_v.at[f].at[d], sem)
            pltpu.async_copy(fm1_hbm.at[f].at[idx_v.at[f]], fm1_v.at[f], sem)

        def drain(f):
            for d in range(D):
                pltpu.make_async_copy(fm2_hbm.at[f * D + d].at[idx_v.at[f]],
                                      fm2_v.at[f].at[d], sem).wait()
                pltpu.make_async_copy(dnn_hbm.at[f * D + d].at[idx_v.at[f]],
                                      dnn_v.at[f].at[d], sem).wait()
            pltpu.make_async_copy(fm1_hbm.at[f].at[idx_v.at[f]],
                                  fm1_v.at[f], sem).wait()

        issue(0)

        def body(f, carry):
            issue(f)
            drain(f - 1)
            return carry

        lax.fori_loop(1, F, body, 0)
        drain(F - 1)

        pltpu.sync_copy(fm1_v, fm1_out.at[:, wid])
        pltpu.sync_copy(fm2_v, fm2_out.at[:, :, wid])
        pltpu.sync_copy(dnn_v, dnn_out.at[:, :, wid])

    return _sc_gather


BM = 512


def _full(shape):
    return pl.BlockSpec(shape, lambda i: tuple(0 for _ in shape))


def kernel(numeric, categorical, fm1_cat, fm1_num_w, fm1_num_b, fm2_tables,
           dnn_tables, num_w, num_b,
           W0, b0, g0, be0, rm0, rv0,
           W1, b1, g1, be1, rm1, rv1,
           W2, b2, g2, be2, rm2, rv2,
           Wo, bo):
    fm2_t = fm2_tables.transpose(0, 2, 1).reshape(F * D, V)
    dnn_t = dnn_tables.transpose(0, 2, 1).reshape(F * D, V)
    cat_t = categorical.T

    fm1T, fm2T4, dnnT4 = _make_sc_gather()(cat_t, fm1_cat, fm2_t, dnn_t)
    fm1T = fm1T.reshape(F, B)
    fm2T = fm2T4.reshape(F * D, B)
    dnnT = dnnT4.reshape(F * D, B)
    numT = numeric.T

    eps = 1e-5

    def fold(W, b, g, be, rm, rv):
        sc = g / jnp.sqrt(rv + eps)
        return W.T * sc[None, :], ((b - rm) * sc + be)[None, :]

    W0T, b0f = fold(W0, b0, g0, be0, rm0, rv0)
    W1T, b1f = fold(W1, b1, g1, be1, rm1, rv1)
    W2T, b2f = fold(W2, b2, g2, be2, rm2, rv2)
    W0Td = W0T[:F * D, :]
    W0Tn = W0T[F * D:, :]
    nwT = num_w.T
    nbf = num_b[None, :]
    wo0 = Wo[0, 0]
    nvec = fm1_num_w.T * wo0
    woh = Wo[0, 2:][:, None]
    params = jnp.stack([wo0, Wo[0, 1], bo[0] + wo0 * fm1_num_b[0]])[None, :]

    def tc_body(dnn_ref, fm2_ref, fm1_ref, num_ref,
                w0d_ref, w0n_ref, b0_ref, w1_ref, b1_ref, w2_ref, b2_ref,
                nw_ref, nb_ref, nvec_ref, woh_ref, par_ref, out_ref):
        cdim0 = (((0,), (0,)), ((), ()))
        fm1s = jnp.sum(fm1_ref[...], axis=0)
        fm2 = fm2_ref[...]
        s = jnp.sum(fm2, axis=0)
        ss = jnp.sum(fm2 * fm2, axis=0)
        fm2nd = 0.5 * (s * s - ss)
        xT = num_ref[...]
        ne = lax.dot_general(xT, nw_ref[...], cdim0,
                             preferred_element_type=jnp.float32) + nb_ref[...]
        h = (lax.dot_general(dnn_ref[...], w0d_ref[...], cdim0,
                             preferred_element_type=jnp.float32)
             + jnp.dot(ne, w0n_ref[...], preferred_element_type=jnp.float32)
             + b0_ref[...])
        h = jnp.maximum(h, 0.0)
        h = jnp.maximum(jnp.dot(h, w1_ref[...],
                                preferred_element_type=jnp.float32) + b1_ref[...], 0.0)
        h = jnp.maximum(jnp.dot(h, w2_ref[...],
                                preferred_element_type=jnp.float32) + b2_ref[...], 0.0)
        lin = (jnp.dot(h, woh_ref[...], preferred_element_type=jnp.float32)[:, 0]
               + lax.dot_general(xT, nvec_ref[...], cdim0,
                                 preferred_element_type=jnp.float32)[:, 0])
        logit = par_ref[0, 0] * fm1s + par_ref[0, 1] * fm2nd + lin + par_ref[0, 2]
        out_ref[...] = 1.0 / (1.0 + jnp.exp(-logit))

    grid = (B // BM,)
    out = pl.pallas_call(
        tc_body,
        grid=grid,
        in_specs=[
            pl.BlockSpec((F * D, BM), lambda i: (0, i)),
            pl.BlockSpec((F * D, BM), lambda i: (0, i)),
            pl.BlockSpec((F, BM), lambda i: (0, i)),
            pl.BlockSpec((NUM, BM), lambda i: (0, i)),
            _full(W0Td.shape), _full(W0Tn.shape), _full(b0f.shape),
            _full(W1T.shape), _full(b1f.shape),
            _full(W2T.shape), _full(b2f.shape),
            _full(nwT.shape), _full(nbf.shape),
            _full(nvec.shape), _full(woh.shape), _full(params.shape),
        ],
        out_specs=pl.BlockSpec((BM,), lambda i: (i,)),
        out_shape=jax.ShapeDtypeStruct((B,), jnp.float32),
    )(dnnT, fm2T, fm1T, numT,
      W0Td, W0Tn, b0f, W1T, b1f, W2T, b2f, nwT, nbf, nvec, woh, params)
    return out


# speedup vs baseline: 4.4751x; 1.5406x over previous
import functools

import jax
import jax.numpy as jnp
from jax import lax
from jax.experimental import pallas as pl
from jax.experimental.pallas import tpu as pltpu
from jax.experimental.pallas import tpu_sc as plsc

B = 4096
F = 26
V = 100000
D = 16
NUM = 13

NC = 2
NS = 16
NW = NC * NS
BPW = B // NW

VT = 784
VP = VT * 128


def _detile(x, rows_out):
    rows = x.shape[0]

    def body(in_ref, out_ref):
        xv = in_ref[...]
        padded = jnp.concatenate(
            [xv, jnp.zeros((8, VP - V), dtype=xv.dtype)], axis=1)
        out_ref[...] = padded.reshape(8, VT, 128)

    grid = (rows_out // 8,)
    return pl.pallas_call(
        body,
        grid=grid,
        in_specs=[pl.BlockSpec((8, V), lambda i: (i, 0))],
        out_specs=pl.BlockSpec((8, VT, 128), lambda i: (i, 0, 0)),
        out_shape=jax.ShapeDtypeStruct((rows_out, VT, 128), jnp.float32),
    )(x)


@functools.lru_cache(maxsize=1)
def _make_sc_gather():
    mesh = plsc.VectorSubcoreMesh(core_axis_name="c", subcore_axis_name="s")

    @functools.partial(
        pl.kernel,
        out_type=[
            jax.ShapeDtypeStruct((F, NW, BPW), jnp.float32),
            jax.ShapeDtypeStruct((F, D, NW, BPW), jnp.float32),
            jax.ShapeDtypeStruct((F, D, NW, BPW), jnp.float32),
        ],
        mesh=mesh,
        scratch_types=[
            pltpu.VMEM((F, BPW), jnp.int32),
            pltpu.VMEM((F, BPW), jnp.float32),
            pltpu.VMEM((F, D, BPW), jnp.float32),
            pltpu.VMEM((F, D, BPW), jnp.float32),
            pltpu.SemaphoreType.DMA,
        ],
        compiler_params=pltpu.CompilerParams(use_tc_tiling_on_sc=False),
    )
    def _sc_gather(cat_hbm, fm1_hbm, fm2_hbm, dnn_hbm,
                   fm1_out, fm2_out, dnn_out,
                   idx_v, fm1_v, fm2_v, dnn_v, sem):
        wid = lax.axis_index("s") * NC + lax.axis_index("c")
        pltpu.sync_copy(cat_hbm.at[:, pl.ds(wid * BPW, BPW)], idx_v)

        def issue(f):
            for d in range(D):
                pltpu.async_copy(fm2_hbm.at[f * D + d].at[idx_v.at[f]],
                                 fm2_v.at[f].at[d], sem)
                pltpu.async_copy(dnn_hbm.at[f * D + d].at[idx_v.at[f]],
                                 dnn_v.at[f].at[d], sem)
            pltpu.async_copy(fm1_hbm.at[f].at[idx_v.at[f]], fm1_v.at[f], sem)

        def drain(f):
            for d in range(D):
                pltpu.make_async_copy(fm2_hbm.at[f * D + d].at[idx_v.at[f]],
                                      fm2_v.at[f].at[d], sem).wait()
                pltpu.make_async_copy(dnn_hbm.at[f * D + d].at[idx_v.at[f]],
                                      dnn_v.at[f].at[d], sem).wait()
            pltpu.make_async_copy(fm1_hbm.at[f].at[idx_v.at[f]],
                                  fm1_v.at[f], sem).wait()

        issue(0)

        def body(f, carry):
            issue(f)
            drain(f - 1)
            return carry

        lax.fori_loop(1, F, body, 0)
        drain(F - 1)

        pltpu.sync_copy(fm1_v, fm1_out.at[:, wid])
        pltpu.sync_copy(fm2_v, fm2_out.at[:, :, wid])
        pltpu.sync_copy(dnn_v, dnn_out.at[:, :, wid])

    return _sc_gather


BM = 512


def _full(shape):
    return pl.BlockSpec(shape, lambda i: tuple(0 for _ in shape))


def kernel(numeric, categorical, fm1_cat, fm1_num_w, fm1_num_b, fm2_tables,
           dnn_tables, num_w, num_b,
           W0, b0, g0, be0, rm0, rv0,
           W1, b1, g1, be1, rm1, rv1,
           W2, b2, g2, be2, rm2, rv2,
           Wo, bo):
    fm2_t = fm2_tables.transpose(0, 2, 1).reshape(F * D, V)
    dnn_t = dnn_tables.transpose(0, 2, 1).reshape(F * D, V)
    cat_t = categorical.T

    fm2_d = _detile(fm2_t, F * D).reshape(F * D, VP)
    dnn_d = _detile(dnn_t, F * D).reshape(F * D, VP)
    fm1_d = _detile(jnp.pad(fm1_cat, ((0, 6), (0, 0))), 32).reshape(32, VP)

    fm1T, fm2T4, dnnT4 = _make_sc_gather()(cat_t, fm1_d, fm2_d, dnn_d)
    fm1T = fm1T.reshape(F, B)
    fm2T = fm2T4.reshape(F * D, B)
    dnnT = dnnT4.reshape(F * D, B)
    numT = numeric.T

    eps = 1e-5

    def fold(W, b, g, be, rm, rv):
        sc = g / jnp.sqrt(rv + eps)
        return W.T * sc[None, :], ((b - rm) * sc + be)[None, :]

    W0T, b0f = fold(W0, b0, g0, be0, rm0, rv0)
    W1T, b1f = fold(W1, b1, g1, be1, rm1, rv1)
    W2T, b2f = fold(W2, b2, g2, be2, rm2, rv2)
    W0Td = W0T[:F * D, :]
    W0Tn = W0T[F * D:, :]
    nwT = num_w.T
    nbf = num_b[None, :]
    wo0 = Wo[0, 0]
    nvec = fm1_num_w.T * wo0
    woh = Wo[0, 2:][:, None]
    params = jnp.stack([wo0, Wo[0, 1], bo[0] + wo0 * fm1_num_b[0]])[None, :]

    def tc_body(dnn_ref, fm2_ref, fm1_ref, num_ref,
                w0d_ref, w0n_ref, b0_ref, w1_ref, b1_ref, w2_ref, b2_ref,
                nw_ref, nb_ref, nvec_ref, woh_ref, par_ref, out_ref):
        cdim0 = (((0,), (0,)), ((), ()))
        fm1s = jnp.sum(fm1_ref[...], axis=0)
        fm2 = fm2_ref[...]
        s = jnp.sum(fm2, axis=0)
        ss = jnp.sum(fm2 * fm2, axis=0)
        fm2nd = 0.5 * (s * s - ss)
        xT = num_ref[...]
        ne = lax.dot_general(xT, nw_ref[...], cdim0,
                             preferred_element_type=jnp.float32) + nb_ref[...]
        h = (lax.dot_general(dnn_ref[...], w0d_ref[...], cdim0,
                             preferred_element_type=jnp.float32)
             + jnp.dot(ne, w0n_ref[...], preferred_element_type=jnp.float32)
             + b0_ref[...])
        h = jnp.maximum(h, 0.0)
        h = jnp.maximum(jnp.dot(h, w1_ref[...],
                                preferred_element_type=jnp.float32) + b1_ref[...], 0.0)
        h = jnp.maximum(jnp.dot(h, w2_ref[...],
                                preferred_element_type=jnp.float32) + b2_ref[...], 0.0)
        lin = (jnp.dot(h, woh_ref[...], preferred_element_type=jnp.float32)[:, 0]
               + lax.dot_general(xT, nvec_ref[...], cdim0,
                                 preferred_element_type=jnp.float32)[:, 0])
        logit = par_ref[0, 0] * fm1s + par_ref[0, 1] * fm2nd + lin + par_ref[0, 2]
        out_ref[...] = 1.0 / (1.0 + jnp.exp(-logit))

    grid = (B // BM,)
    out = pl.pallas_call(
        tc_body,
        grid=grid,
        in_specs=[
            pl.BlockSpec((F * D, BM), lambda i: (0, i)),
            pl.BlockSpec((F * D, BM), lambda i: (0, i)),
            pl.BlockSpec((F, BM), lambda i: (0, i)),
            pl.BlockSpec((NUM, BM), lambda i: (0, i)),
            _full(W0Td.shape), _full(W0Tn.shape), _full(b0f.shape),
            _full(W1T.shape), _full(b1f.shape),
            _full(W2T.shape), _full(b2f.shape),
            _full(nwT.shape), _full(nbf.shape),
            _full(nvec.shape), _full(woh.shape), _full(params.shape),
        ],
        out_specs=pl.BlockSpec((BM,), lambda i: (i,)),
        out_shape=jax.ShapeDtypeStruct((B,), jnp.float32),
    )(dnnT, fm2T, fm1T, numT,
      W0Td, W0Tn, b0f, W1T, b1f, W2T, b2f, nwT, nbf, nvec, woh, params)
    return out


# speedup vs baseline: 4.5988x; 1.0276x over previous
import functools

import jax
import jax.numpy as jnp
from jax import lax
from jax.experimental import pallas as pl
from jax.experimental.pallas import tpu as pltpu
from jax.experimental.pallas import tpu_sc as plsc

B = 4096
F = 26
V = 100000
D = 16
NUM = 13

NC = 2
NS = 16
NW = NC * NS
BPW = B // NW

VT = 784
VP = VT * 128


def _detile(x, rows_out):
    rows = x.shape[0]

    def body(in_ref, out_ref):
        xv = in_ref[...]
        padded = jnp.concatenate(
            [xv, jnp.zeros((8, VP - V), dtype=xv.dtype)], axis=1)
        out_ref[...] = padded.reshape(8, VT, 128)

    grid = (rows_out // 8,)
    return pl.pallas_call(
        body,
        grid=grid,
        in_specs=[pl.BlockSpec((8, V), lambda i: (i, 0))],
        out_specs=pl.BlockSpec((8, VT, 128), lambda i: (i, 0, 0)),
        out_shape=jax.ShapeDtypeStruct((rows_out, VT, 128), jnp.float32),
    )(x)


@functools.lru_cache(maxsize=2)
def _make_sc_gather(with_fm1):
    mesh = plsc.VectorSubcoreMesh(core_axis_name="c", subcore_axis_name="s")

    out_type = [jax.ShapeDtypeStruct((F, D, NW, BPW), jnp.float32)]
    scratch = [
        pltpu.VMEM((F, BPW), jnp.int32),
        pltpu.VMEM((F, D, BPW), jnp.float32),
        pltpu.SemaphoreType.DMA,
    ]
    if with_fm1:
        out_type = [jax.ShapeDtypeStruct((F, NW, BPW), jnp.float32)] + out_type
        scratch.insert(1, pltpu.VMEM((F, BPW), jnp.float32))

    @functools.partial(
        pl.kernel,
        out_type=out_type,
        mesh=mesh,
        scratch_types=scratch,
        compiler_params=pltpu.CompilerParams(use_tc_tiling_on_sc=False),
    )
    def _sc_gather(cat_hbm, *rest):
        if with_fm1:
            (fm1_hbm, tab_hbm, fm1_out, tab_out,
             idx_v, fm1_v, tab_v, sem) = rest
        else:
            tab_hbm, tab_out, idx_v, tab_v, sem = rest
        wid = lax.axis_index("s") * NC + lax.axis_index("c")
        pltpu.sync_copy(cat_hbm.at[:, pl.ds(wid * BPW, BPW)], idx_v)

        def issue(f):
            for d in range(D):
                pltpu.async_copy(tab_hbm.at[f * D + d].at[idx_v.at[f]],
                                 tab_v.at[f].at[d], sem)
            if with_fm1:
                pltpu.async_copy(fm1_hbm.at[f].at[idx_v.at[f]], fm1_v.at[f], sem)

        def drain(f):
            for d in range(D):
                pltpu.make_async_copy(tab_hbm.at[f * D + d].at[idx_v.at[f]],
                                      tab_v.at[f].at[d], sem).wait()
            if with_fm1:
                pltpu.make_async_copy(fm1_hbm.at[f].at[idx_v.at[f]],
                                      fm1_v.at[f], sem).wait()

        issue(0)

        def body(f, carry):
            issue(f)
            drain(f - 1)
            return carry

        lax.fori_loop(1, F, body, 0)
        drain(F - 1)

        if with_fm1:
            pltpu.sync_copy(fm1_v, fm1_out.at[:, wid])
        pltpu.sync_copy(tab_v, tab_out.at[:, :, wid])

    return _sc_gather


BM = 512


def _full(shape):
    return pl.BlockSpec(shape, lambda i: tuple(0 for _ in shape))


def kernel(numeric, categorical, fm1_cat, fm1_num_w, fm1_num_b, fm2_tables,
           dnn_tables, num_w, num_b,
           W0, b0, g0, be0, rm0, rv0,
           W1, b1, g1, be1, rm1, rv1,
           W2, b2, g2, be2, rm2, rv2,
           Wo, bo):
    fm2_t = fm2_tables.transpose(0, 2, 1).reshape(F * D, V)
    dnn_t = dnn_tables.transpose(0, 2, 1).reshape(F * D, V)
    cat_t = categorical.T

    fm2_d = _detile(fm2_t, F * D).reshape(F * D, VP)
    dnn_d = _detile(dnn_t, F * D).reshape(F * D, VP)
    fm1_d = _detile(jnp.pad(fm1_cat, ((0, 6), (0, 0))), 32).reshape(32, VP)

    fm1T, fm2T4 = _make_sc_gather(True)(cat_t, fm1_d, fm2_d)
    (dnnT4,) = _make_sc_gather(False)(cat_t, dnn_d)
    fm1T = fm1T.reshape(F, B)
    fm2T = fm2T4.reshape(F * D, B)
    dnnT = dnnT4.reshape(F * D, B)
    numT = numeric.T

    eps = 1e-5

    def fold(W, b, g, be, rm, rv):
        sc = g / jnp.sqrt(rv + eps)
        return W.T * sc[None, :], ((b - rm) * sc + be)[None, :]

    W0T, b0f = fold(W0, b0, g0, be0, rm0, rv0)
    W1T, b1f = fold(W1, b1, g1, be1, rm1, rv1)
    W2T, b2f = fold(W2, b2, g2, be2, rm2, rv2)
    W0Td = W0T[:F * D, :]
    W0Tn = W0T[F * D:, :]
    nwT = num_w.T
    nbf = num_b[None, :]
    wo0 = Wo[0, 0]
    nvec = fm1_num_w.T * wo0
    woh = Wo[0, 2:][:, None]
    params = jnp.stack([wo0, Wo[0, 1], bo[0] + wo0 * fm1_num_b[0]])[None, :]

    def tc_body(dnn_ref, fm2_ref, fm1_ref, num_ref,
                w0d_ref, w0n_ref, b0_ref, w1_ref, b1_ref, w2_ref, b2_ref,
                nw_ref, nb_ref, nvec_ref, woh_ref, par_ref, out_ref):
        cdim0 = (((0,), (0,)), ((), ()))
        fm1s = jnp.sum(fm1_ref[...], axis=0)
        fm2 = fm2_ref[...]
        s = jnp.sum(fm2, axis=0)
        ss = jnp.sum(fm2 * fm2, axis=0)
        fm2nd = 0.5 * (s * s - ss)
        xT = num_ref[...]
        ne = lax.dot_general(xT, nw_ref[...], cdim0,
                             preferred_element_type=jnp.float32) + nb_ref[...]
        h = (lax.dot_general(dnn_ref[...], w0d_ref[...], cdim0,
                             preferred_element_type=jnp.float32)
             + jnp.dot(ne, w0n_ref[...], preferred_element_type=jnp.float32)
             + b0_ref[...])
        h = jnp.maximum(h, 0.0)
        h = jnp.maximum(jnp.dot(h, w1_ref[...],
                                preferred_element_type=jnp.float32) + b1_ref[...], 0.0)
        h = jnp.maximum(jnp.dot(h, w2_ref[...],
                                preferred_element_type=jnp.float32) + b2_ref[...], 0.0)
        lin = (jnp.dot(h, woh_ref[...], preferred_element_type=jnp.float32)[:, 0]
               + lax.dot_general(xT, nvec_ref[...], cdim0,
                                 preferred_element_type=jnp.float32)[:, 0])
        logit = par_ref[0, 0] * fm1s + par_ref[0, 1] * fm2nd + lin + par_ref[0, 2]
        out_ref[...] = 1.0 / (1.0 + jnp.exp(-logit))

    grid = (B // BM,)
    out = pl.pallas_call(
        tc_body,
        grid=grid,
        in_specs=[
            pl.BlockSpec((F * D, BM), lambda i: (0, i)),
            pl.BlockSpec((F * D, BM), lambda i: (0, i)),
            pl.BlockSpec((F, BM), lambda i: (0, i)),
            pl.BlockSpec((NUM, BM), lambda i: (0, i)),
            _full(W0Td.shape), _full(W0Tn.shape), _full(b0f.shape),
            _full(W1T.shape), _full(b1f.shape),
            _full(W2T.shape), _full(b2f.shape),
            _full(nwT.shape), _full(nbf.shape),
            _full(nvec.shape), _full(woh.shape), _full(params.shape),
        ],
        out_specs=pl.BlockSpec((BM,), lambda i: (i,)),
        out_shape=jax.ShapeDtypeStruct((B,), jnp.float32),
    )(dnnT, fm2T, fm1T, numT,
      W0Td, W0Tn, b0f, W1T, b1f, W2T, b2f, nwT, nbf, nvec, woh, params)
    return out


# speedup vs baseline: 5.0531x; 1.0988x over previous
import functools

import jax
import jax.numpy as jnp
from jax import lax
from jax.experimental import pallas as pl
from jax.experimental.pallas import tpu as pltpu
from jax.experimental.pallas import tpu_sc as plsc

B = 4096
F = 26
V = 100000
D = 16
NUM = 13

NC = 2
NS = 16
NW = NC * NS
BPW = B // NW

VT = 784
VP = VT * 128


def _detile(x, rows_out, row0=0):
    def body(in_ref, out_ref):
        xv = in_ref[...]
        padded = jnp.concatenate(
            [xv, jnp.zeros((8, VP - V), dtype=xv.dtype)], axis=1)
        out_ref[...] = padded.reshape(8, VT, 128)

    blk0 = row0 // 8
    grid = (rows_out // 8,)
    return pl.pallas_call(
        body,
        grid=grid,
        in_specs=[pl.BlockSpec((8, V), lambda i: (i + blk0, 0))],
        out_specs=pl.BlockSpec((8, VT, 128), lambda i: (i, 0, 0)),
        out_shape=jax.ShapeDtypeStruct((rows_out, VT, 128), jnp.float32),
    )(x)


FH = F // 2


@functools.lru_cache(maxsize=4)
def _make_sc_gather(with_fm1, f0):
    mesh = plsc.VectorSubcoreMesh(core_axis_name="c", subcore_axis_name="s")

    out_type = [jax.ShapeDtypeStruct((FH, D, NW, BPW), jnp.float32)]
    scratch = [
        pltpu.VMEM((FH, BPW), jnp.int32),
        pltpu.VMEM((FH, D, BPW), jnp.float32),
        pltpu.SemaphoreType.DMA,
    ]
    if with_fm1:
        out_type = [jax.ShapeDtypeStruct((FH, NW, BPW), jnp.float32)] + out_type
        scratch.insert(1, pltpu.VMEM((FH, BPW), jnp.float32))

    @functools.partial(
        pl.kernel,
        out_type=out_type,
        mesh=mesh,
        scratch_types=scratch,
        compiler_params=pltpu.CompilerParams(use_tc_tiling_on_sc=False),
    )
    def _sc_gather(cat_hbm, *rest):
        if with_fm1:
            (fm1_hbm, tab_hbm, fm1_out, tab_out,
             idx_v, fm1_v, tab_v, sem) = rest
        else:
            tab_hbm, tab_out, idx_v, tab_v, sem = rest
        wid = lax.axis_index("s") * NC + lax.axis_index("c")
        pltpu.sync_copy(cat_hbm.at[pl.ds(f0, FH), pl.ds(wid * BPW, BPW)], idx_v)

        def issue(f):
            for d in range(D):
                pltpu.async_copy(tab_hbm.at[f * D + d].at[idx_v.at[f]],
                                 tab_v.at[f].at[d], sem)
            if with_fm1:
                pltpu.async_copy(fm1_hbm.at[f0 + f].at[idx_v.at[f]],
                                 fm1_v.at[f], sem)

        def drain(f):
            for d in range(D):
                pltpu.make_async_copy(tab_hbm.at[f * D + d].at[idx_v.at[f]],
                                      tab_v.at[f].at[d], sem).wait()
            if with_fm1:
                pltpu.make_async_copy(fm1_hbm.at[f0 + f].at[idx_v.at[f]],
                                      fm1_v.at[f], sem).wait()

        issue(0)

        def body(f, carry):
            issue(f)
            drain(f - 1)
            return carry

        lax.fori_loop(1, FH, body, 0)
        drain(FH - 1)

        if with_fm1:
            pltpu.sync_copy(fm1_v, fm1_out.at[:, wid])
        pltpu.sync_copy(tab_v, tab_out.at[:, :, wid])

    return _sc_gather


BM = 512


def _full(shape):
    return pl.BlockSpec(shape, lambda i: tuple(0 for _ in shape))


def kernel(numeric, categorical, fm1_cat, fm1_num_w, fm1_num_b, fm2_tables,
           dnn_tables, num_w, num_b,
           W0, b0, g0, be0, rm0, rv0,
           W1, b1, g1, be1, rm1, rv1,
           W2, b2, g2, be2, rm2, rv2,
           Wo, bo):
    fm2_t = fm2_tables.transpose(0, 2, 1).reshape(F * D, V)
    dnn_t = dnn_tables.transpose(0, 2, 1).reshape(F * D, V)
    cat_t = categorical.T

    HD = FH * D
    fm1_d = _detile(fm1_cat, 32).reshape(32, VP)
    fm2_da = _detile(fm2_t, HD, 0).reshape(HD, VP)
    fm1T_a, fm2T_a = _make_sc_gather(True, 0)(cat_t, fm1_d, fm2_da)
    fm2_db = _detile(fm2_t, HD, HD).reshape(HD, VP)
    fm1T_b, fm2T_b = _make_sc_gather(True, FH)(cat_t, fm1_d, fm2_db)
    dnn_da = _detile(dnn_t, HD, 0).reshape(HD, VP)
    (dnnT_a,) = _make_sc_gather(False, 0)(cat_t, dnn_da)
    dnn_db = _detile(dnn_t, HD, HD).reshape(HD, VP)
    (dnnT_b,) = _make_sc_gather(False, FH)(cat_t, dnn_db)

    fm1T_a = fm1T_a.reshape(FH, B)
    fm1T_b = fm1T_b.reshape(FH, B)
    fm2T_a = fm2T_a.reshape(HD, B)
    fm2T_b = fm2T_b.reshape(HD, B)
    dnnT_a = dnnT_a.reshape(HD, B)
    dnnT_b = dnnT_b.reshape(HD, B)
    numT = numeric.T

    eps = 1e-5

    def fold(W, b, g, be, rm, rv):
        sc = g / jnp.sqrt(rv + eps)
        return W.T * sc[None, :], ((b - rm) * sc + be)[None, :]

    W0T, b0f = fold(W0, b0, g0, be0, rm0, rv0)
    W1T, b1f = fold(W1, b1, g1, be1, rm1, rv1)
    W2T, b2f = fold(W2, b2, g2, be2, rm2, rv2)
    W0Td = W0T[:F * D, :]
    W0Tn = W0T[F * D:, :]
    nwT = num_w.T
    nbf = num_b[None, :]
    wo0 = Wo[0, 0]
    nvec = fm1_num_w.T * wo0
    woh = Wo[0, 2:][:, None]
    params = jnp.stack([wo0, Wo[0, 1], bo[0] + wo0 * fm1_num_b[0]])[None, :]

    def tc_body(dnna_ref, dnnb_ref, fm2a_ref, fm2b_ref, fm1a_ref, fm1b_ref,
                num_ref,
                w0d_ref, w0n_ref, b0_ref, w1_ref, b1_ref, w2_ref, b2_ref,
                nw_ref, nb_ref, nvec_ref, woh_ref, par_ref, out_ref):
        cdim0 = (((0,), (0,)), ((), ()))
        fm1s = jnp.sum(fm1a_ref[...], axis=0) + jnp.sum(fm1b_ref[...], axis=0)
        fm2 = jnp.concatenate([fm2a_ref[...], fm2b_ref[...]], axis=0)
        s = jnp.sum(fm2, axis=0)
        ss = jnp.sum(fm2 * fm2, axis=0)
        fm2nd = 0.5 * (s * s - ss)
        xT = num_ref[...]
        ne = lax.dot_general(xT, nw_ref[...], cdim0,
                             preferred_element_type=jnp.float32) + nb_ref[...]
        dnn = jnp.concatenate([dnna_ref[...], dnnb_ref[...]], axis=0)
        h = (lax.dot_general(dnn, w0d_ref[...], cdim0,
                             preferred_element_type=jnp.float32)
             + jnp.dot(ne, w0n_ref[...], preferred_element_type=jnp.float32)
             + b0_ref[...])
        h = jnp.maximum(h, 0.0)
        h = jnp.maximum(jnp.dot(h, w1_ref[...],
                                preferred_element_type=jnp.float32) + b1_ref[...], 0.0)
        h = jnp.maximum(jnp.dot(h, w2_ref[...],
                                preferred_element_type=jnp.float32) + b2_ref[...], 0.0)
        lin = (jnp.dot(h, woh_ref[...], preferred_element_type=jnp.float32)[:, 0]
               + lax.dot_general(xT, nvec_ref[...], cdim0,
                                 preferred_element_type=jnp.float32)[:, 0])
        logit = par_ref[0, 0] * fm1s + par_ref[0, 1] * fm2nd + lin + par_ref[0, 2]
        out_ref[...] = 1.0 / (1.0 + jnp.exp(-logit))

    grid = (B // BM,)
    out = pl.pallas_call(
        tc_body,
        grid=grid,
        in_specs=[
            pl.BlockSpec((HD, BM), lambda i: (0, i)),
            pl.BlockSpec((HD, BM), lambda i: (0, i)),
            pl.BlockSpec((HD, BM), lambda i: (0, i)),
            pl.BlockSpec((HD, BM), lambda i: (0, i)),
            pl.BlockSpec((FH, BM), lambda i: (0, i)),
            pl.BlockSpec((FH, BM), lambda i: (0, i)),
            pl.BlockSpec((NUM, BM), lambda i: (0, i)),
            _full(W0Td.shape), _full(W0Tn.shape), _full(b0f.shape),
            _full(W1T.shape), _full(b1f.shape),
            _full(W2T.shape), _full(b2f.shape),
            _full(nwT.shape), _full(nbf.shape),
            _full(nvec.shape), _full(woh.shape), _full(params.shape),
        ],
        out_specs=pl.BlockSpec((BM,), lambda i: (i,)),
        out_shape=jax.ShapeDtypeStruct((B,), jnp.float32),
    )(dnnT_a, dnnT_b, fm2T_a, fm2T_b, fm1T_a, fm1T_b, numT,
      W0Td, W0Tn, b0f, W1T, b1f, W2T, b2f, nwT, nbf, nvec, woh, params)
    return out
